# Initial kernel scaffold; baseline (speedup 1.0000x reference)
import jax
import jax.numpy as jnp
from jax.experimental import pallas as pl


def kernel(x, edge_index, node_pair, W1, b1, W2, b2, Wfc, bfc):
    raise NotImplementedError("write your pallas kernel here")


# speedup vs baseline: 18.5577x; 18.5577x over previous
import dataclasses
import functools

import jax
import jax.numpy as jnp
from jax import lax
from jax.experimental import pallas as pl
from jax.experimental.pallas import tpu as pltpu
from jax.experimental.pallas import tpu_sc as plsc

N = 10000
E = 320000
D = 128
H = 128

NCORE = 2
NSUB = 16
NTILE = NCORE * NSUB
EPT = E // NTILE
CH = 80
NCHUNK = EPT // CH
N_PAD = 10240

_mesh = plsc.VectorSubcoreMesh(core_axis_name="c", subcore_axis_name="s")

_cp = pltpu.CompilerParams()
if "needs_layout_passes" in pltpu.CompilerParams.__dataclass_fields__:
    _cp = dataclasses.replace(_cp, needs_layout_passes=False)


@functools.partial(
    pl.kernel,
    out_type=jax.ShapeDtypeStruct((NTILE, 1, N_PAD), jnp.float32),
    mesh=_mesh,
    compiler_params=_cp,
    scratch_types=[
        pltpu.VMEM((NCHUNK, 1, CH), jnp.int32),
        pltpu.VMEM((1, N_PAD), jnp.float32),
        pltpu.SemaphoreType.DMA,
    ],
)
def _sc_degree(dst_hbm, out_hbm, idx_v, hist_v, sem):
    c = lax.axis_index("c")
    s = lax.axis_index("s")
    g = c * NSUB + s

    pltpu.async_copy(dst_hbm.at[g], idx_v, sem).wait()

    @pl.loop(0, N_PAD // 16)
    def _(i):
        hist_v[0, pl.ds(i * 16, 16)] = jnp.zeros((16,), jnp.float32)

    zeros16 = jnp.zeros((16,), jnp.int32)
    ones16 = jnp.ones((16,), jnp.float32)

    @pl.loop(0, NCHUNK)
    def _(j):
        @pl.loop(0, CH // 16)
        def _(k):
            idx = idx_v[j, 0, pl.ds(k * 16, 16)]
            plsc.addupdate_scatter(hist_v, [zeros16, idx], ones16)

    pltpu.sync_copy(hist_v, out_hbm.at[g])


@functools.partial(
    pl.kernel,
    out_type=jax.ShapeDtypeStruct((NCORE, N_PAD, H), jnp.float32),
    mesh=_mesh,
    compiler_params=_cp,
    scratch_types=[
        pltpu.VMEM((NCHUNK, 1, CH), jnp.int32),
        pltpu.VMEM((NCHUNK, 1, CH), jnp.int32),
        pltpu.VMEM((CH,), jnp.int32),
        pltpu.VMEM((CH,), jnp.int32),
        pltpu.VMEM((CH, H), jnp.float32),
        pltpu.VMEM_SHARED((N_PAD, H), jnp.float32),
        pltpu.SemaphoreType.DMA,
    ],
)
def _sc_edge_pass(src_hbm, dst_hbm, y_hbm, zeros_hbm, out_hbm,
                  idx_s, idx_d, is1, id1, rows_v, acc_sh, sem):
    c = lax.axis_index("c")
    s = lax.axis_index("s")
    g = c * NSUB + s

    @pl.when(s == 0)
    def _():
        pltpu.sync_copy(zeros_hbm, acc_sh)

    pltpu.async_copy(src_hbm.at[g], idx_s, sem).wait()
    pltpu.async_copy(dst_hbm.at[g], idx_d, sem).wait()
    plsc.subcore_barrier()

    @pl.loop(0, NCHUNK)
    def _(j):
        @pl.loop(0, CH // 16)
        def _(k):
            sl = pl.ds(k * 16, 16)
            is1[sl] = idx_s[j, 0, sl]
            id1[sl] = idx_d[j, 0, sl]
        pltpu.async_copy(y_hbm.at[is1], rows_v, sem).wait()
        pltpu.sync_copy(rows_v, acc_sh.at[id1], add=True)

    plsc.subcore_barrier()

    @pl.when(s == 0)
    def _():
        pltpu.sync_copy(acc_sh, out_hbm.at[c])


BR = 400


def _tc0_body(degp_ref, dis_ref):
    deg = jnp.sum(degp_ref[...], axis=0) + 1.0
    dis_ref[...] = lax.rsqrt(deg)


def _tc1_body(x_ref, w_ref, dis_ref, y_ref):
    xw = jnp.dot(x_ref[...], w_ref[...], preferred_element_type=jnp.float32)
    y_ref[...] = xw * dis_ref[...]


def _tc2_body(p_ref, y_ref, dis_ref, b_ref, w_ref, out_ref):
    dis = dis_ref[...]
    pre = (p_ref[0] + p_ref[1] + y_ref[...]) * dis + b_ref[...]
    h = jnp.maximum(pre, 0.0)
    hw = jnp.dot(h, w_ref[...], preferred_element_type=jnp.float32)
    out_ref[...] = hw * dis


def _tc3_body(p_ref, y_ref, dis_ref, b_ref, out_ref):
    dis = dis_ref[...]
    pre = (p_ref[0] + p_ref[1] + y_ref[...]) * dis + b_ref[...]
    out_ref[...] = jnp.maximum(pre, 0.0)


def _tcf_body(np_ref, h_ref, w_ref, b_ref, out_ref):
    i0 = np_ref[0]
    i1 = np_ref[1]
    r0 = h_ref[pl.ds(i0, 1), :]
    r1 = h_ref[pl.ds(i1, 1), :]
    val = (jnp.sum(r0 * w_ref[0:1, :]) + jnp.sum(r1 * w_ref[1:2, :])
           + b_ref[0, 0])
    out_ref[...] = jnp.broadcast_to(jax.nn.sigmoid(val), (1, 1))


def kernel(x, edge_index, node_pair, W1, b1, W2, b2, Wfc, bfc):
    src = edge_index[0].reshape(NTILE, NCHUNK, 1, CH)
    dst = edge_index[1].reshape(NTILE, NCHUNK, 1, CH)
    b1r = b1.reshape(1, H)
    b2r = b2.reshape(1, H)
    wfc = Wfc.reshape(2, H)
    bfc_r = bfc.reshape(1, 1)
    zeros = jnp.zeros((N_PAD, H), jnp.float32)

    degp = _sc_degree(dst)

    dis_row = pl.pallas_call(
        _tc0_body,
        in_specs=[pl.BlockSpec((NTILE, 1, N_PAD), lambda: (0, 0, 0))],
        out_specs=pl.BlockSpec((1, N_PAD), lambda: (0, 0)),
        out_shape=jax.ShapeDtypeStruct((1, N_PAD), jnp.float32),
    )(degp)
    dis = dis_row.reshape(N_PAD)[:N].reshape(N, 1)

    grid = (N // BR,)
    mat_spec = pl.BlockSpec((BR, H), lambda i: (i, 0))
    w_spec = pl.BlockSpec((D, H), lambda i: (0, 0))
    b_spec = pl.BlockSpec((1, H), lambda i: (0, 0))
    dis_spec = pl.BlockSpec((BR, 1), lambda i: (i, 0))
    p_spec = pl.BlockSpec((NCORE, BR, H), lambda i: (0, i, 0))

    y1 = pl.pallas_call(
        _tc1_body,
        grid=grid,
        in_specs=[mat_spec, w_spec, dis_spec],
        out_specs=mat_spec,
        out_shape=jax.ShapeDtypeStruct((N, H), jnp.float32),
    )(x, W1, dis)

    p1 = _sc_edge_pass(src, dst, y1, zeros)

    y2 = pl.pallas_call(
        _tc2_body,
        grid=grid,
        in_specs=[p_spec, mat_spec, dis_spec, b_spec, w_spec],
        out_specs=mat_spec,
        out_shape=jax.ShapeDtypeStruct((N, H), jnp.float32),
    )(p1, y1, dis, b1r, W2)

    p2 = _sc_edge_pass(src, dst, y2, zeros)

    h2 = pl.pallas_call(
        _tc3_body,
        grid=grid,
        in_specs=[p_spec, mat_spec, dis_spec, b_spec],
        out_specs=mat_spec,
        out_shape=jax.ShapeDtypeStruct((N, H), jnp.float32),
    )(p2, y2, dis, b2r)

    out = pl.pallas_call(
        _tcf_body,
        in_specs=[
            pl.BlockSpec(memory_space=pltpu.SMEM),
            pl.BlockSpec((N, H), lambda: (0, 0)),
            pl.BlockSpec((2, H), lambda: (0, 0)),
            pl.BlockSpec((1, 1), lambda: (0, 0)),
        ],
        out_specs=pl.BlockSpec((1, 1), lambda: (0, 0)),
        out_shape=jax.ShapeDtypeStruct((1, 1), jnp.float32),
    )(node_pair, h2, wfc, bfc_r)

    return out.reshape(1)


# speedup vs baseline: 24.5972x; 1.3254x over previous
import dataclasses
import functools

import jax
import jax.numpy as jnp
from jax import lax
from jax.experimental import pallas as pl
from jax.experimental.pallas import tpu as pltpu
from jax.experimental.pallas import tpu_sc as plsc

N = 10000
E = 320000
D = 128
H = 128

NCORE = 2
NSUB = 16
NTILE = NCORE * NSUB
EPT = E // NTILE
CH = 80
NCHUNK = EPT // CH
N_PAD = 10240

_mesh = plsc.VectorSubcoreMesh(core_axis_name="c", subcore_axis_name="s")

_cp = pltpu.CompilerParams()
if "needs_layout_passes" in pltpu.CompilerParams.__dataclass_fields__:
    _cp = dataclasses.replace(_cp, needs_layout_passes=False)


@functools.partial(
    pl.kernel,
    out_type=jax.ShapeDtypeStruct((NTILE, 1, N_PAD), jnp.float32),
    mesh=_mesh,
    compiler_params=_cp,
    scratch_types=[
        pltpu.VMEM((NCHUNK, 1, CH), jnp.int32),
        pltpu.VMEM((1, N_PAD), jnp.float32),
        pltpu.SemaphoreType.DMA,
    ],
)
def _sc_degree(dst_hbm, out_hbm, idx_v, hist_v, sem):
    c = lax.axis_index("c")
    s = lax.axis_index("s")
    g = c * NSUB + s

    pltpu.async_copy(dst_hbm.at[g], idx_v, sem).wait()

    @pl.loop(0, N_PAD // 16)
    def _(i):
        hist_v[0, pl.ds(i * 16, 16)] = jnp.zeros((16,), jnp.float32)

    zeros16 = jnp.zeros((16,), jnp.int32)
    ones16 = jnp.ones((16,), jnp.float32)

    @pl.loop(0, NCHUNK)
    def _(j):
        @pl.loop(0, CH // 16)
        def _(k):
            idx = idx_v[j, 0, pl.ds(k * 16, 16)]
            plsc.addupdate_scatter(hist_v, [zeros16, idx], ones16)

    pltpu.sync_copy(hist_v, out_hbm.at[g])


@functools.partial(
    pl.kernel,
    out_type=jax.ShapeDtypeStruct((NCORE, N_PAD, H), jnp.float32),
    mesh=_mesh,
    compiler_params=_cp,
    scratch_types=[
        pltpu.VMEM((CH,), jnp.int32),
        pltpu.VMEM((CH,), jnp.int32),
        pltpu.VMEM((CH,), jnp.int32),
        pltpu.VMEM((CH,), jnp.int32),
        pltpu.VMEM((CH, H), jnp.float32),
        pltpu.VMEM((CH, H), jnp.float32),
        pltpu.SemaphoreType.DMA,
        pltpu.SemaphoreType.DMA,
        pltpu.SemaphoreType.DMA,
        pltpu.SemaphoreType.DMA,
        pltpu.SemaphoreType.DMA,
        pltpu.SemaphoreType.DMA,
        pltpu.VMEM_SHARED((N_PAD, H), jnp.float32),
    ],
)
def _sc_edge_pass(src_hbm, dst_hbm, y_hbm, zeros_hbm, out_hbm,
                  is0, id0, is1, id1, rows0, rows1,
                  semg0, semg1, semis0, semid0, semis1, semid1, acc_sh):
    c = lax.axis_index("c")
    s = lax.axis_index("s")
    g = c * NSUB + s

    @pl.when(s == 0)
    def _():
        pltpu.sync_copy(zeros_hbm, acc_sh)

    def ld(arr, j, buf, sem):
        pltpu.async_copy(arr.at[g, j, 0], buf, sem)

    def ldwait(buf, sem):
        pltpu.make_async_copy(src_hbm.at[g, 0, 0], buf, sem).wait()

    ld(src_hbm, 0, is0, semis0)
    ld(dst_hbm, 0, id0, semid0)
    ld(src_hbm, 1, is1, semis1)
    ld(dst_hbm, 1, id1, semid1)

    g0 = pltpu.make_async_copy(y_hbm.at[is0], rows0, semg0)
    g1 = pltpu.make_async_copy(y_hbm.at[is1], rows1, semg1)
    ldwait(is0, semis0)
    g0.start()

    NP = (NCHUNK - 1) // 2

    @pl.loop(0, NP)
    def _(t):
        j0 = 2 * t
        g0.wait()
        ldwait(is1, semis1)
        g1.start()
        ld(src_hbm, j0 + 2, is0, semis0)
        ldwait(id0, semid0)
        pltpu.sync_copy(rows0, acc_sh.at[id0], add=True)
        ld(dst_hbm, j0 + 2, id0, semid0)
        g1.wait()
        ldwait(is0, semis0)
        g0.start()

        @pl.when(t < NP - 1)
        def _():
            ld(src_hbm, j0 + 3, is1, semis1)

        ldwait(id1, semid1)
        pltpu.sync_copy(rows1, acc_sh.at[id1], add=True)

        @pl.when(t < NP - 1)
        def _():
            ld(dst_hbm, j0 + 3, id1, semid1)

    g0.wait()
    ldwait(id0, semid0)
    pltpu.sync_copy(rows0, acc_sh.at[id0], add=True)

    plsc.subcore_barrier()

    @pl.when(s == 0)
    def _():
        pltpu.sync_copy(acc_sh, out_hbm.at[c])


BR = 400


def _tc0_body(degp_ref, dis_ref):
    deg = jnp.sum(degp_ref[...], axis=0) + 1.0
    dis_ref[...] = lax.rsqrt(deg).reshape(N_PAD, 1)


def _tc1_body(x_ref, w_ref, dis_ref, y_ref):
    xw = jnp.dot(x_ref[...], w_ref[...], preferred_element_type=jnp.float32)
    y_ref[...] = xw * dis_ref[...]


def _tc2_body(p_ref, y_ref, dis_ref, b_ref, w_ref, out_ref):
    dis = dis_ref[...]
    pre = (p_ref[0] + p_ref[1] + y_ref[...]) * dis + b_ref[...]
    h = jnp.maximum(pre, 0.0)
    hw = jnp.dot(h, w_ref[...], preferred_element_type=jnp.float32)
    out_ref[...] = hw * dis


def _tcf_body(np_ref, p_ref, y_ref, dis_ref, b2_ref, w_ref, b_ref, out_ref):
    def row(i):
        pre = ((p_ref[0, pl.ds(i, 1), :] + p_ref[1, pl.ds(i, 1), :]
                + y_ref[pl.ds(i, 1), :]) * dis_ref[pl.ds(i, 1), :]
               + b2_ref[...])
        return jnp.maximum(pre, 0.0)

    r0 = row(np_ref[0])
    r1 = row(np_ref[1])
    val = (jnp.sum(r0 * w_ref[0:1, :]) + jnp.sum(r1 * w_ref[1:2, :])
           + b_ref[0, 0])
    out_ref[...] = jnp.broadcast_to(jax.nn.sigmoid(val), (1, 1))


def kernel(x, edge_index, node_pair, W1, b1, W2, b2, Wfc, bfc):
    src = edge_index[0].reshape(NTILE, NCHUNK, 1, CH)
    dst = edge_index[1].reshape(NTILE, NCHUNK, 1, CH)
    b1r = b1.reshape(1, H)
    b2r = b2.reshape(1, H)
    wfc = Wfc.reshape(2, H)
    bfc_r = bfc.reshape(1, 1)
    zeros = jnp.zeros((N_PAD, H), jnp.float32)

    degp = _sc_degree(dst)

    dis = pl.pallas_call(
        _tc0_body,
        in_specs=[pl.BlockSpec((NTILE, 1, N_PAD), lambda: (0, 0, 0))],
        out_specs=pl.BlockSpec((N_PAD, 1), lambda: (0, 0)),
        out_shape=jax.ShapeDtypeStruct((N_PAD, 1), jnp.float32),
    )(degp)

    grid = (N // BR,)
    mat_spec = pl.BlockSpec((BR, H), lambda i: (i, 0))
    w_spec = pl.BlockSpec((D, H), lambda i: (0, 0))
    b_spec = pl.BlockSpec((1, H), lambda i: (0, 0))
    dis_spec = pl.BlockSpec((BR, 1), lambda i: (i, 0))
    p_spec = pl.BlockSpec((NCORE, BR, H), lambda i: (0, i, 0))

    y1 = pl.pallas_call(
        _tc1_body,
        grid=grid,
        in_specs=[mat_spec, w_spec, dis_spec],
        out_specs=mat_spec,
        out_shape=jax.ShapeDtypeStruct((N, H), jnp.float32),
    )(x, W1, dis)

    p1 = _sc_edge_pass(src, dst, y1, zeros)

    y2 = pl.pallas_call(
        _tc2_body,
        grid=grid,
        in_specs=[p_spec, mat_spec, dis_spec, b_spec, w_spec],
        out_specs=mat_spec,
        out_shape=jax.ShapeDtypeStruct((N, H), jnp.float32),
    )(p1, y1, dis, b1r, W2)

    p2 = _sc_edge_pass(src, dst, y2, zeros)

    out = pl.pallas_call(
        _tcf_body,
        in_specs=[
            pl.BlockSpec(memory_space=pltpu.SMEM),
            pl.BlockSpec((NCORE, N_PAD, H), lambda: (0, 0, 0)),
            pl.BlockSpec((N, H), lambda: (0, 0)),
            pl.BlockSpec((N_PAD, 1), lambda: (0, 0)),
            pl.BlockSpec((1, H), lambda: (0, 0)),
            pl.BlockSpec((2, H), lambda: (0, 0)),
            pl.BlockSpec((1, 1), lambda: (0, 0)),
        ],
        out_specs=pl.BlockSpec((1, 1), lambda: (0, 0)),
        out_shape=jax.ShapeDtypeStruct((1, 1), jnp.float32),
    )(node_pair, p2, y2, dis, b2r, wfc, bfc_r)

    return out.reshape(1)


# speedup vs baseline: 30.6963x; 1.2480x over previous
import dataclasses
import functools

import jax
import jax.numpy as jnp
from jax import lax
from jax.experimental import pallas as pl
from jax.experimental.pallas import tpu as pltpu
from jax.experimental.pallas import tpu_sc as plsc

N = 10000
E = 320000
D = 128
H = 128

NCORE = 2
NSUB = 16
NTILE = NCORE * NSUB
EPT = E // NTILE
CH = 80
NCHUNK = EPT // CH
N_PAD = 10240

_mesh = plsc.VectorSubcoreMesh(core_axis_name="c", subcore_axis_name="s")

_cp = pltpu.CompilerParams()
if "needs_layout_passes" in pltpu.CompilerParams.__dataclass_fields__:
    _cp = dataclasses.replace(_cp, needs_layout_passes=False)


@functools.partial(
    pl.kernel,
    out_type=jax.ShapeDtypeStruct((NTILE, 1, N_PAD), jnp.float32),
    mesh=_mesh,
    compiler_params=_cp,
    scratch_types=[
        pltpu.VMEM((NCHUNK, 1, CH), jnp.int32),
        pltpu.VMEM((1, N_PAD), jnp.float32),
        pltpu.SemaphoreType.DMA,
    ],
)
def _sc_degree(dst_hbm, out_hbm, idx_v, hist_v, sem):
    c = lax.axis_index("c")
    s = lax.axis_index("s")
    g = c * NSUB + s

    pltpu.async_copy(dst_hbm.at[g], idx_v, sem).wait()

    @pl.loop(0, N_PAD // 16)
    def _(i):
        hist_v[0, pl.ds(i * 16, 16)] = jnp.zeros((16,), jnp.float32)

    zeros16 = jnp.zeros((16,), jnp.int32)
    ones16 = jnp.ones((16,), jnp.float32)

    @pl.loop(0, NCHUNK)
    def _(j):
        @pl.loop(0, CH // 16)
        def _(k):
            idx = idx_v[j, 0, pl.ds(k * 16, 16)]
            plsc.addupdate_scatter(hist_v, [zeros16, idx], ones16)

    pltpu.sync_copy(hist_v, out_hbm.at[g])


@functools.partial(
    pl.kernel,
    out_type=jax.ShapeDtypeStruct((NCORE, N_PAD, H), jnp.float32),
    mesh=_mesh,
    compiler_params=_cp,
    scratch_types=[
        pltpu.VMEM((4, CH), jnp.int32),
        pltpu.VMEM((4, CH), jnp.int32),
        pltpu.VMEM((4, CH, H), jnp.float32),
        pltpu.SemaphoreType.DMA((4,)),
        pltpu.SemaphoreType.DMA((4,)),
        pltpu.SemaphoreType.DMA((4,)),
        pltpu.SemaphoreType.DMA((4,)),
        pltpu.VMEM_SHARED((N_PAD, H), jnp.float32),
    ],
)
def _sc_edge_pass(src_hbm, dst_hbm, y_hbm, zeros_hbm, out_hbm,
                  is_v, id_v, rows_v, semg, semsc, semis, semid, acc_sh):
    c = lax.axis_index("c")
    s = lax.axis_index("s")
    g = c * NSUB + s

    @pl.when(s == 0)
    def _():
        pltpu.sync_copy(zeros_hbm, acc_sh)

    def ld_is(b, j):
        pltpu.async_copy(src_hbm.at[g, j, 0], is_v.at[b], semis.at[b])

    def ld_id(b, j):
        pltpu.async_copy(dst_hbm.at[g, j, 0], id_v.at[b], semid.at[b])

    def wait_is(b):
        pltpu.make_async_copy(src_hbm.at[g, 0, 0], is_v.at[b], semis.at[b]).wait()

    def wait_id(b):
        pltpu.make_async_copy(dst_hbm.at[g, 0, 0], id_v.at[b], semid.at[b]).wait()

    def gather(b):
        pltpu.async_copy(y_hbm.at[is_v.at[b]], rows_v.at[b], semg.at[b])

    def wait_gather(b):
        pltpu.make_async_copy(y_hbm.at[is_v.at[b]], rows_v.at[b],
                              semg.at[b]).wait()

    def scat(b):
        pltpu.async_copy(rows_v.at[b], acc_sh.at[id_v.at[b]], semsc.at[b],
                         add=True)

    def wait_scat(b):
        pltpu.make_async_copy(rows_v.at[b], acc_sh.at[id_v.at[b]],
                              semsc.at[b]).wait()

    for b in range(4):
        ld_is(b, b)
    ld_id(0, 0)
    ld_id(1, 1)
    wait_is(0)
    gather(0)
    wait_is(1)
    gather(1)

    def step(j, b, b2):
        wait_gather(b)

        @pl.when(j + 4 < NCHUNK)
        def _():
            ld_is(b, j + 4)

        wait_id(b)
        scat(b)

        @pl.when(j >= 2)
        def _():
            wait_scat(b2)

        @pl.when(j + 2 < NCHUNK)
        def _():
            ld_id(b2, j + 2)
            wait_is(b2)
            gather(b2)

    @pl.loop(0, NCHUNK // 4)
    def _(t):
        for b in range(4):
            step(4 * t + b, b, (b + 2) % 4)

    step(NCHUNK - 1, (NCHUNK - 1) % 4, (NCHUNK + 1) % 4)

    wait_scat((NCHUNK - 2) % 4)
    wait_scat((NCHUNK - 1) % 4)

    plsc.subcore_barrier()

    @pl.when(s == 0)
    def _():
        pltpu.sync_copy(acc_sh, out_hbm.at[c])


BR = 400


def _tc0_body(degp_ref, dis_ref):
    deg = jnp.sum(degp_ref[...], axis=0) + 1.0
    dis_ref[...] = lax.rsqrt(deg).reshape(N_PAD, 1)


def _tc1_body(x_ref, w_ref, dis_ref, y_ref):
    xw = jnp.dot(x_ref[...], w_ref[...], preferred_element_type=jnp.float32)
    y_ref[...] = xw * dis_ref[...]


def _tc2_body(p_ref, y_ref, dis_ref, b_ref, w_ref, out_ref):
    dis = dis_ref[...]
    pre = (p_ref[0] + p_ref[1] + y_ref[...]) * dis + b_ref[...]
    h = jnp.maximum(pre, 0.0)
    hw = jnp.dot(h, w_ref[...], preferred_element_type=jnp.float32)
    out_ref[...] = hw * dis


def _tcf_body(np_ref, p_ref, y_ref, dis_ref, b2_ref, w_ref, b_ref, out_ref):
    def row(i):
        pre = ((p_ref[0, pl.ds(i, 1), :] + p_ref[1, pl.ds(i, 1), :]
                + y_ref[pl.ds(i, 1), :]) * dis_ref[pl.ds(i, 1), :]
               + b2_ref[...])
        return jnp.maximum(pre, 0.0)

    r0 = row(np_ref[0])
    r1 = row(np_ref[1])
    val = (jnp.sum(r0 * w_ref[0:1, :]) + jnp.sum(r1 * w_ref[1:2, :])
           + b_ref[0, 0])
    out_ref[...] = jnp.broadcast_to(jax.nn.sigmoid(val), (1, 1))


def kernel(x, edge_index, node_pair, W1, b1, W2, b2, Wfc, bfc):
    src = edge_index[0].reshape(NTILE, NCHUNK, 1, CH)
    dst = edge_index[1].reshape(NTILE, NCHUNK, 1, CH)
    b1r = b1.reshape(1, H)
    b2r = b2.reshape(1, H)
    wfc = Wfc.reshape(2, H)
    bfc_r = bfc.reshape(1, 1)
    zeros = jnp.zeros((N_PAD, H), jnp.float32)

    degp = _sc_degree(dst)

    dis = pl.pallas_call(
        _tc0_body,
        in_specs=[pl.BlockSpec((NTILE, 1, N_PAD), lambda: (0, 0, 0))],
        out_specs=pl.BlockSpec((N_PAD, 1), lambda: (0, 0)),
        out_shape=jax.ShapeDtypeStruct((N_PAD, 1), jnp.float32),
    )(degp)

    grid = (N // BR,)
    mat_spec = pl.BlockSpec((BR, H), lambda i: (i, 0))
    w_spec = pl.BlockSpec((D, H), lambda i: (0, 0))
    b_spec = pl.BlockSpec((1, H), lambda i: (0, 0))
    dis_spec = pl.BlockSpec((BR, 1), lambda i: (i, 0))
    p_spec = pl.BlockSpec((NCORE, BR, H), lambda i: (0, i, 0))

    y1 = pl.pallas_call(
        _tc1_body,
        grid=grid,
        in_specs=[mat_spec, w_spec, dis_spec],
        out_specs=mat_spec,
        out_shape=jax.ShapeDtypeStruct((N, H), jnp.float32),
    )(x, W1, dis)

    p1 = _sc_edge_pass(src, dst, y1, zeros)

    y2 = pl.pallas_call(
        _tc2_body,
        grid=grid,
        in_specs=[p_spec, mat_spec, dis_spec, b_spec, w_spec],
        out_specs=mat_spec,
        out_shape=jax.ShapeDtypeStruct((N, H), jnp.float32),
    )(p1, y1, dis, b1r, W2)

    p2 = _sc_edge_pass(src, dst, y2, zeros)

    out = pl.pallas_call(
        _tcf_body,
        in_specs=[
            pl.BlockSpec(memory_space=pltpu.SMEM),
            pl.BlockSpec((NCORE, N_PAD, H), lambda: (0, 0, 0)),
            pl.BlockSpec((N, H), lambda: (0, 0)),
            pl.BlockSpec((N_PAD, 1), lambda: (0, 0)),
            pl.BlockSpec((1, H), lambda: (0, 0)),
            pl.BlockSpec((2, H), lambda: (0, 0)),
            pl.BlockSpec((1, 1), lambda: (0, 0)),
        ],
        out_specs=pl.BlockSpec((1, 1), lambda: (0, 0)),
        out_shape=jax.ShapeDtypeStruct((1, 1), jnp.float32),
    )(node_pair, p2, y2, dis, b2r, wfc, bfc_r)

    return out.reshape(1)
